# baseline (speedup 1.0000x reference)
import functools

import jax
import jax.numpy as jnp
from jax.experimental import pallas as pl
from jax.experimental.pallas import tpu as pltpu

HEADS = 4
TOPK = 3
NEG_SLOPE = 0.2


def _cosnorm(x):
    return x / jnp.clip(jnp.linalg.norm(x, axis=-1, keepdims=True), 1e-8)


def _topk_mask(sim, L):
    cols = [sim[:, j:j + 1] for j in range(L)]
    outs = []
    for j in range(L):
        r = jnp.zeros_like(cols[0])
        for k in range(L):
            if k == j:
                continue
            if k < j:
                beat = cols[k] >= cols[j]
            else:
                beat = cols[k] > cols[j]
            r = r + beat.astype(jnp.float32)
        outs.append((r < (TOPK - 0.5)).astype(jnp.float32))
    return jnp.concatenate(outs, axis=1)


def _fused_kernel(t_ref, l_ref, i_ref, sl_ref, si_ref,
                  w0_ref, as0_ref, ad0_ref, b0_ref, g0_ref, be0_ref,
                  w1_ref, as1_ref, ad1_ref, b1_ref, g1_ref, be1_ref,
                  w2_ref, as2_ref, ad2_ref, b2_ref, g2_ref, be2_ref,
                  o_ref, a_scr, an_scr, *, T, L, I, H):
    N = T + L + I
    out_ch = H // HEADS
    t = t_ref[0]
    lab = l_ref[0]
    img = i_ref[0]
    dn = (((1,), (1,)), ((), ()))
    mask_l = _topk_mask(sl_ref[0], L)
    mask_i = _topk_mask(si_ref[0], I)
    eye_l = (jax.lax.broadcasted_iota(jnp.int32, (L, L), 0)
             == jax.lax.broadcasted_iota(jnp.int32, (L, L), 1)).astype(jnp.float32)
    mask_lT = jax.lax.dot_general(eye_l, mask_l, dn,
                                  preferred_element_type=jnp.float32)
    eye_i = (jax.lax.broadcasted_iota(jnp.int32, (I, I), 0)
             == jax.lax.broadcasted_iota(jnp.int32, (I, I), 1)).astype(jnp.float32)
    mask_iT = jax.lax.dot_general(eye_i, mask_i, dn,
                                  preferred_element_type=jnp.float32)
    r = jax.lax.broadcasted_iota(jnp.int32, (T, T), 0)
    c = jax.lax.broadcasted_iota(jnp.int32, (T, T), 1)
    chain = ((r - c == 1) | (c - r == 1) | (r == c)).astype(jnp.float32)
    ones_ll = jnp.ones((L, L), jnp.float32)
    ones_ii = jnp.ones((I, I), jnp.float32)
    a_scr[0:T, 0:T] = chain
    a_scr[0:T, T:T + L] = mask_l
    a_scr[0:T, T + L:N] = mask_i
    a_scr[T:T + L, 0:T] = mask_lT
    a_scr[T:T + L, T:T + L] = ones_ll + eye_l
    a_scr[T:T + L, T + L:N] = jnp.ones((L, I), jnp.float32)
    a_scr[T + L:N, 0:T] = mask_iT
    a_scr[T + L:N, T:T + L] = jnp.ones((I, L), jnp.float32)
    a_scr[T + L:N, T + L:N] = ones_ii + eye_i
    A = a_scr[...]
    an_scr[...] = jnp.where(A > 0, 0.0, -1e30)
    Aneg = an_scr[...]

    x = jnp.concatenate([t, lab, img], axis=0)
    plist = [
        (w0_ref, as0_ref, ad0_ref, b0_ref, g0_ref, be0_ref),
        (w1_ref, as1_ref, ad1_ref, b1_ref, g1_ref, be1_ref),
        (w2_ref, as2_ref, ad2_ref, b2_ref, g2_ref, be2_ref),
    ]
    for (w_ref, as_ref, ad_ref, b_ref, g_ref, be_ref) in plist:
        h = jnp.dot(x.astype(jnp.bfloat16), w_ref[...],
                    preferred_element_type=jnp.float32)
        aggs = []
        for hd in range(HEADS):
            hh = h[:, hd * out_ch:(hd + 1) * out_ch]
            asr = as_ref[hd:hd + 1, :]
            adr = ad_ref[hd:hd + 1, :]
            a_s_row = jax.lax.dot_general(
                asr, hh, dn, preferred_element_type=jnp.float32)
            a_d_col = jax.lax.dot_general(
                hh, adr, dn, preferred_element_type=jnp.float32)
            alpha = a_d_col + a_s_row
            alpha = jnp.maximum(alpha, NEG_SLOPE * alpha)
            malpha = alpha + Aneg
            amax = jnp.max(malpha, axis=1, keepdims=True)
            ex = A * jnp.exp(malpha - amax)
            rden = 1.0 / (jnp.sum(ex, axis=1, keepdims=True) + 1e-16)
            aggs.append(jnp.dot(ex, hh,
                                preferred_element_type=jnp.float32) * rden)
        agg = jnp.concatenate(aggs, axis=1)
        out = jnp.maximum(agg + b_ref[...], 0.0)
        y = out + x
        mu = jnp.mean(y, axis=1, keepdims=True)
        var = jnp.maximum(jnp.mean(y * y, axis=1, keepdims=True) - mu * mu,
                          0.0)
        x = (y - mu) / jnp.sqrt(var + 1e-5) * g_ref[...] + be_ref[...]
    o_ref[0] = x[0:T, :]


def _run_fused(text_repr, label_repr, image_repr, params, interpret=False):
    B, T, H = text_repr.shape
    L = label_repr.shape[1]
    I = image_repr.shape[1]
    N = T + L + I
    out_ch = H // HEADS
    tn = _cosnorm(text_repr)
    sim_l = jnp.einsum('bth,blh->btl', tn, _cosnorm(label_repr))
    sim_i = jnp.einsum('bth,bih->bti', tn, _cosnorm(image_repr))
    wspec = pl.BlockSpec((H, H), lambda bb: (0, 0))
    aspec = pl.BlockSpec((HEADS, out_ch), lambda b: (0, 0))
    vspec = pl.BlockSpec((1, H), lambda b: (0, 0))
    in_specs = [
        pl.BlockSpec((1, T, H), lambda b: (b, 0, 0)),
        pl.BlockSpec((1, L, H), lambda b: (b, 0, 0)),
        pl.BlockSpec((1, I, H), lambda b: (b, 0, 0)),
        pl.BlockSpec((1, T, L), lambda b: (b, 0, 0)),
        pl.BlockSpec((1, T, I), lambda b: (b, 0, 0)),
    ]
    args = [text_repr, label_repr, image_repr, sim_l, sim_i]
    for (W, a_s, a_d, b, g, be) in params:
        in_specs += [wspec, aspec, aspec, vspec, vspec, vspec]
        args += [W.astype(jnp.bfloat16), a_s, a_d, b.reshape(1, -1),
                 g.reshape(1, -1), be.reshape(1, -1)]
    return pl.pallas_call(
        functools.partial(_fused_kernel, T=T, L=L, I=I, H=H),
        grid=(B,),
        in_specs=in_specs,
        out_specs=pl.BlockSpec((1, T, H), lambda b: (b, 0, 0)),
        out_shape=jax.ShapeDtypeStruct((B, T, H), jnp.float32),
        scratch_shapes=[pltpu.VMEM((N, N), jnp.float32),
                        pltpu.VMEM((N, N), jnp.float32)],
        interpret=interpret,
    )(*args)


def kernel(text_repr, label_repr, image_repr,
           W0, as0, ad0, b0, g0, be0,
           W1, as1, ad1, b1, g1, be1,
           W2, as2, ad2, b2, g2, be2):
    params = [
        (W0, as0, ad0, b0, g0, be0),
        (W1, as1, ad1, b1, g1, be1),
        (W2, as2, ad2, b2, g2, be2),
    ]
    return _run_fused(text_repr, label_repr, image_repr, params)


# speedup vs baseline: 1.0628x; 1.0628x over previous
import functools

import jax
import jax.numpy as jnp
from jax.experimental import pallas as pl
from jax.experimental.pallas import tpu as pltpu

HEADS = 4
TOPK = 3
NEG_SLOPE = 0.2


def _cosnorm(x):
    return x / jnp.clip(jnp.linalg.norm(x, axis=-1, keepdims=True), 1e-8)


def _topk_mask(sim, L):
    cols = [sim[:, j:j + 1] for j in range(L)]
    outs = []
    for j in range(L):
        r = jnp.zeros_like(cols[0])
        for k in range(L):
            if k == j:
                continue
            if k < j:
                beat = cols[k] >= cols[j]
            else:
                beat = cols[k] > cols[j]
            r = r + beat.astype(jnp.float32)
        outs.append((r < (TOPK - 0.5)).astype(jnp.float32))
    return jnp.concatenate(outs, axis=1)


def _fused_kernel(t_ref, l_ref, i_ref, sl_ref, si_ref,
                  w0_ref, as0_ref, ad0_ref, b0_ref, g0_ref, be0_ref,
                  w1_ref, as1_ref, ad1_ref, b1_ref, g1_ref, be1_ref,
                  w2_ref, as2_ref, ad2_ref, b2_ref, g2_ref, be2_ref,
                  o_ref, a_scr, an_scr, *, T, L, I, H):
    N = T + L + I
    out_ch = H // HEADS
    t = t_ref[0]
    lab = l_ref[0]
    img = i_ref[0]
    dn = (((1,), (1,)), ((), ()))
    mask_l = _topk_mask(sl_ref[0], L)
    mask_i = _topk_mask(si_ref[0], I)
    eye_l = (jax.lax.broadcasted_iota(jnp.int32, (L, L), 0)
             == jax.lax.broadcasted_iota(jnp.int32, (L, L), 1)).astype(jnp.float32)
    mask_lT = jax.lax.dot_general(eye_l, mask_l, dn,
                                  preferred_element_type=jnp.float32)
    eye_i = (jax.lax.broadcasted_iota(jnp.int32, (I, I), 0)
             == jax.lax.broadcasted_iota(jnp.int32, (I, I), 1)).astype(jnp.float32)
    mask_iT = jax.lax.dot_general(eye_i, mask_i, dn,
                                  preferred_element_type=jnp.float32)
    r = jax.lax.broadcasted_iota(jnp.int32, (T, T), 0)
    c = jax.lax.broadcasted_iota(jnp.int32, (T, T), 1)
    chain = ((r - c == 1) | (c - r == 1) | (r == c)).astype(jnp.float32)
    ones_ll = jnp.ones((L, L), jnp.float32)
    ones_ii = jnp.ones((I, I), jnp.float32)
    a_scr[0:T, 0:T] = chain
    a_scr[0:T, T:T + L] = mask_l
    a_scr[0:T, T + L:N] = mask_i
    a_scr[T:T + L, 0:T] = mask_lT
    a_scr[T:T + L, T:T + L] = ones_ll + eye_l
    a_scr[T:T + L, T + L:N] = jnp.ones((L, I), jnp.float32)
    a_scr[T + L:N, 0:T] = mask_iT
    a_scr[T + L:N, T:T + L] = jnp.ones((I, L), jnp.float32)
    a_scr[T + L:N, T + L:N] = ones_ii + eye_i
    A = a_scr[...]
    an_scr[...] = jnp.where(A > 0, 0.0, -1e30)
    Aneg = an_scr[...]

    x = jnp.concatenate([t, lab, img], axis=0)
    plist = [
        (w0_ref, as0_ref, ad0_ref, b0_ref, g0_ref, be0_ref),
        (w1_ref, as1_ref, ad1_ref, b1_ref, g1_ref, be1_ref),
        (w2_ref, as2_ref, ad2_ref, b2_ref, g2_ref, be2_ref),
    ]
    for (w_ref, as_ref, ad_ref, b_ref, g_ref, be_ref) in plist:
        h = jnp.dot(x, w_ref[...], preferred_element_type=jnp.float32)
        aggs = []
        for hd in range(HEADS):
            hh = h[:, hd * out_ch:(hd + 1) * out_ch]
            asr = as_ref[hd:hd + 1, :]
            adr = ad_ref[hd:hd + 1, :]
            a_s_row = jax.lax.dot_general(
                asr, hh, dn, preferred_element_type=jnp.float32)
            a_d_col = jax.lax.dot_general(
                hh, adr, dn, preferred_element_type=jnp.float32)
            alpha = a_d_col + a_s_row
            alpha = jnp.maximum(alpha, NEG_SLOPE * alpha)
            malpha = alpha + Aneg
            amax = jnp.max(malpha, axis=1, keepdims=True)
            ex = A * jnp.exp(malpha - amax)
            rden = 1.0 / (jnp.sum(ex, axis=1, keepdims=True) + 1e-16)
            aggs.append(jnp.dot(ex, hh,
                                preferred_element_type=jnp.float32) * rden)
        agg = jnp.concatenate(aggs, axis=1)
        out = jnp.maximum(agg + b_ref[...], 0.0)
        y = out + x
        mu = jnp.mean(y, axis=1, keepdims=True)
        var = jnp.maximum(jnp.mean(y * y, axis=1, keepdims=True) - mu * mu,
                          0.0)
        x = (y - mu) / jnp.sqrt(var + 1e-5) * g_ref[...] + be_ref[...]
    o_ref[0] = x[0:T, :]


def _run_fused(text_repr, label_repr, image_repr, params, interpret=False):
    B, T, H = text_repr.shape
    L = label_repr.shape[1]
    I = image_repr.shape[1]
    N = T + L + I
    out_ch = H // HEADS
    tn = _cosnorm(text_repr)
    sim_l = jnp.einsum('bth,blh->btl', tn, _cosnorm(label_repr))
    sim_i = jnp.einsum('bth,bih->bti', tn, _cosnorm(image_repr))
    wspec = pl.BlockSpec((H, H), lambda bb: (0, 0))
    aspec = pl.BlockSpec((HEADS, out_ch), lambda b: (0, 0))
    vspec = pl.BlockSpec((1, H), lambda b: (0, 0))
    in_specs = [
        pl.BlockSpec((1, T, H), lambda b: (b, 0, 0)),
        pl.BlockSpec((1, L, H), lambda b: (b, 0, 0)),
        pl.BlockSpec((1, I, H), lambda b: (b, 0, 0)),
        pl.BlockSpec((1, T, L), lambda b: (b, 0, 0)),
        pl.BlockSpec((1, T, I), lambda b: (b, 0, 0)),
    ]
    args = [text_repr, label_repr, image_repr, sim_l, sim_i]
    for (W, a_s, a_d, b, g, be) in params:
        in_specs += [wspec, aspec, aspec, vspec, vspec, vspec]
        args += [W, a_s, a_d, b.reshape(1, -1), g.reshape(1, -1),
                 be.reshape(1, -1)]
    return pl.pallas_call(
        functools.partial(_fused_kernel, T=T, L=L, I=I, H=H),
        grid=(B,),
        in_specs=in_specs,
        out_specs=pl.BlockSpec((1, T, H), lambda b: (b, 0, 0)),
        out_shape=jax.ShapeDtypeStruct((B, T, H), jnp.float32),
        scratch_shapes=[pltpu.VMEM((N, N), jnp.float32),
                        pltpu.VMEM((N, N), jnp.float32)],
        interpret=interpret,
    )(*args)


def kernel(text_repr, label_repr, image_repr,
           W0, as0, ad0, b0, g0, be0,
           W1, as1, ad1, b1, g1, be1,
           W2, as2, ad2, b2, g2, be2):
    params = [
        (W0, as0, ad0, b0, g0, be0),
        (W1, as1, ad1, b1, g1, be1),
        (W2, as2, ad2, b2, g2, be2),
    ]
    return _run_fused(text_repr, label_repr, image_repr, params)
